# Initial kernel scaffold; baseline (speedup 1.0000x reference)
import jax
import jax.numpy as jnp
from jax.experimental import pallas as pl


def kernel(in_feat, edge_index, W0, b0, W1, b1):
    raise NotImplementedError("write your pallas kernel here")


# speedup vs baseline: 6.7884x; 6.7884x over previous
import functools

import jax
import jax.numpy as jnp
from jax import lax
from jax.experimental import pallas as pl
from jax.experimental.pallas import tpu as pltpu
from jax.experimental.pallas import tpu_sc as plsc

_N = 10000
_E = 320000
_D = 128
_NPAD = 10240
_NTILES = 32
_EPT = _E // _NTILES
_CHUNK = 80
_NCHUNK = _EPT // _CHUNK
_ROWS_PER_TILE = _NPAD // 16


def _mesh():
    return plsc.VectorSubcoreMesh(core_axis_name="c", subcore_axis_name="s")


def _sc_params():
    return pltpu.CompilerParams(needs_layout_passes=False)


@functools.lru_cache(maxsize=None)
def _deg_kernel():
    @functools.partial(
        pl.kernel,
        out_type=jax.ShapeDtypeStruct((_NTILES, 2, _NPAD), jnp.float32),
        mesh=_mesh(),
        compiler_params=_sc_params(),
        scratch_types=[
            pltpu.VMEM((_EPT,), jnp.int32),
            pltpu.VMEM((_EPT,), jnp.int32),
            pltpu.VMEM((_NPAD,), jnp.float32),
            pltpu.VMEM((_NPAD,), jnp.float32),
        ],
    )
    def deg(src_hbm, dst_hbm, out_hbm, src_v, dst_v, tsrc_v, tdst_v):
        c = lax.axis_index("c")
        s = lax.axis_index("s")
        wid = c * 16 + s
        zero16 = jnp.zeros((16,), jnp.float32)

        def zero_body(i, carry):
            tsrc_v[pl.ds(i * 16, 16)] = zero16
            tdst_v[pl.ds(i * 16, 16)] = zero16
            return carry

        lax.fori_loop(0, _NPAD // 16, zero_body, 0)

        pltpu.sync_copy(src_hbm.at[pl.ds(wid * _EPT, _EPT)], src_v)
        pltpu.sync_copy(dst_hbm.at[pl.ds(wid * _EPT, _EPT)], dst_v)

        ones16 = jnp.ones((16,), jnp.float32)

        def count_body(i, carry):
            si = src_v[pl.ds(i * 16, 16)]
            di = dst_v[pl.ds(i * 16, 16)]
            plsc.addupdate_scatter(tsrc_v, [si], ones16)
            plsc.addupdate_scatter(tdst_v, [di], ones16)
            return carry

        lax.fori_loop(0, _EPT // 16, count_body, 0)

        pltpu.sync_copy(tsrc_v, out_hbm.at[wid, 0])
        pltpu.sync_copy(tdst_v, out_hbm.at[wid, 1])

    return deg


@functools.lru_cache(maxsize=None)
def _agg_kernel():
    @functools.partial(
        pl.kernel,
        out_type=jax.ShapeDtypeStruct((2, _NPAD, _D), jnp.float32),
        mesh=_mesh(),
        compiler_params=_sc_params(),
        scratch_types=[
            pltpu.VMEM((_NCHUNK, _CHUNK), jnp.int32),
            pltpu.VMEM((_NCHUNK, _CHUNK), jnp.int32),
            pltpu.VMEM((_CHUNK, _D), jnp.float32),
            pltpu.VMEM_SHARED((_NPAD, _D), jnp.float32),
            pltpu.SemaphoreType.DMA,
        ],
    )
    def agg(h_hbm, src_hbm, dst_hbm, zeros_hbm, out_hbm,
            src_v, dst_v, rows_v, acc_sh, sem):
        c = lax.axis_index("c")
        s = lax.axis_index("s")
        wid = c * 16 + s
        r0 = s * _ROWS_PER_TILE

        pltpu.sync_copy(zeros_hbm.at[pl.ds(r0, _ROWS_PER_TILE)],
                        acc_sh.at[pl.ds(r0, _ROWS_PER_TILE)])
        pltpu.sync_copy(src_hbm.at[wid], src_v)
        pltpu.sync_copy(dst_hbm.at[wid], dst_v)
        plsc.subcore_barrier()

        def body(i, carry):
            pltpu.async_copy(h_hbm.at[src_v.at[i]], rows_v, sem).wait()
            pltpu.sync_copy(rows_v, acc_sh.at[dst_v.at[i]], add=True)
            return carry

        lax.fori_loop(0, _NCHUNK, body, 0)

        plsc.subcore_barrier()
        pltpu.sync_copy(acc_sh.at[pl.ds(r0, _ROWS_PER_TILE)],
                        out_hbm.at[c, pl.ds(r0, _ROWS_PER_TILE)])

    return agg


_ROWS_BLK = 2048


def _norms(deg_ref):
    d = deg_ref[...]
    out_deg = jnp.sum(d[:, :_NTILES], axis=1, keepdims=True)
    in_deg = jnp.sum(d[:, _NTILES:], axis=1, keepdims=True)
    ns = lax.rsqrt(jnp.maximum(out_deg, 1.0))
    nd = lax.rsqrt(jnp.maximum(in_deg, 1.0))
    return ns, nd


def _tc1_body(deg_ref, x_ref, w_ref, o_ref):
    ns, _ = _norms(deg_ref)
    h = jnp.dot(x_ref[...], w_ref[...], preferred_element_type=jnp.float32)
    o_ref[...] = h * ns


def _tc2_body(deg_ref, a0_ref, a1_ref, b_ref, w_ref, o_ref):
    ns, nd = _norms(deg_ref)
    h = (a0_ref[...] + a1_ref[...]) * nd + b_ref[...]
    h = jnp.dot(h, w_ref[...], preferred_element_type=jnp.float32)
    o_ref[...] = h * ns


def _tc3_body(deg_ref, a0_ref, a1_ref, b_ref, o_ref):
    _, nd = _norms(deg_ref)
    o_ref[...] = (a0_ref[...] + a1_ref[...]) * nd + b_ref[...]


_GRID = _NPAD // _ROWS_BLK

_DEG_SPEC = pl.BlockSpec((_ROWS_BLK, 2 * _NTILES), lambda i: (i, 0))
_MAT_SPEC = pl.BlockSpec((_ROWS_BLK, _D), lambda i: (i, 0))
_W_SPEC = pl.BlockSpec((_D, _D), lambda i: (0, 0))
_B_SPEC = pl.BlockSpec((1, _D), lambda i: (0, 0))
_OUT_TYPE = jax.ShapeDtypeStruct((_NPAD, _D), jnp.float32)


def _tc1(deg, x, w):
    return pl.pallas_call(
        _tc1_body, grid=(_GRID,),
        in_specs=[_DEG_SPEC, _MAT_SPEC, _W_SPEC],
        out_specs=_MAT_SPEC, out_shape=_OUT_TYPE,
    )(deg, x, w)


def _tc2(deg, a0, a1, b, w):
    return pl.pallas_call(
        _tc2_body, grid=(_GRID,),
        in_specs=[_DEG_SPEC, _MAT_SPEC, _MAT_SPEC, _B_SPEC, _W_SPEC],
        out_specs=_MAT_SPEC, out_shape=_OUT_TYPE,
    )(deg, a0, a1, b, w)


def _tc3(deg, a0, a1, b):
    return pl.pallas_call(
        _tc3_body, grid=(_GRID,),
        in_specs=[_DEG_SPEC, _MAT_SPEC, _MAT_SPEC, _B_SPEC],
        out_specs=_MAT_SPEC, out_shape=_OUT_TYPE,
    )(deg, a0, a1, b)


def kernel(in_feat, edge_index, W0, b0, W1, b1):
    ei = edge_index.astype(jnp.int32)
    src = ei[0]
    dst = ei[1]
    src3 = src.reshape(_NTILES, _NCHUNK, _CHUNK)
    dst3 = dst.reshape(_NTILES, _NCHUNK, _CHUNK)

    xp = jnp.zeros((_NPAD, _D), jnp.float32).at[:_N].set(in_feat)
    zeros = jnp.zeros((_NPAD, _D), jnp.float32)
    b0r = b0.reshape(1, _D)
    b1r = b1.reshape(1, _D)

    degp = _deg_kernel()(src, dst)
    deg64 = degp.transpose(1, 0, 2).reshape(2 * _NTILES, _NPAD).T

    h1s = _tc1(deg64, xp, W0)
    m1 = _agg_kernel()(h1s, src3, dst3, zeros)
    h2s = _tc2(deg64, m1[0], m1[1], b0r, W1)
    m2 = _agg_kernel()(h2s, src3, dst3, zeros)
    out = _tc3(deg64, m2[0], m2[1], b1r)
    return out[:_N]
